# baseline (speedup 1.0000x reference)
import functools

import jax
import jax.numpy as jnp
from jax import lax
from jax.experimental import pallas as pl
from jax.experimental.pallas import tpu as pltpu
from jax.experimental.pallas import tpu_sc as plsc

NUM_CORES = 2
NUM_SUBCORES = 16
NUM_WORKERS = NUM_CORES * NUM_SUBCORES
LANES = 16
CHUNK = 128
PACK_BLK = 1000


def _make_pack(rows, dim):
    half = rows // 2
    assert half % PACK_BLK == 0
    grid = half // PACK_BLK

    def body(a_ref, b_ref, out_ref):
        out_ref[:, 0:dim] = a_ref[...]
        out_ref[:, dim:2 * dim] = b_ref[...]

    return pl.pallas_call(
        body,
        grid=(grid,),
        in_specs=[
            pl.BlockSpec((PACK_BLK, dim), lambda i: (i, 0)),
            pl.BlockSpec((PACK_BLK, dim), lambda i: (i + grid, 0)),
        ],
        out_specs=pl.BlockSpec((PACK_BLK, 2 * dim), lambda i: (i, 0)),
        out_shape=jax.ShapeDtypeStruct((half, 2 * dim), jnp.float32),
    )


def _make_gather(batch, half, dim):
    assert batch % (NUM_WORKERS * CHUNK) == 0 and dim % LANES == 0
    b_per_w = batch // NUM_WORKERS
    n_chunks = b_per_w // CHUNK
    col_groups = dim // LANES
    mesh = plsc.VectorSubcoreMesh(core_axis_name="c", subcore_axis_name="s")

    @functools.partial(
        pl.kernel,
        mesh=mesh,
        out_type=jax.ShapeDtypeStruct((batch, dim), jnp.float32),
        scratch_types=[
            pltpu.VMEM((b_per_w,), jnp.int32),
            pltpu.VMEM((n_chunks, CHUNK), jnp.int32),
            pltpu.VMEM((CHUNK, 2 * dim), jnp.float32),
            pltpu.VMEM((b_per_w, dim), jnp.float32),
            pltpu.SemaphoreType.DMA,
        ],
    )
    def k(table_hbm, idx_hbm, out_hbm, idx_v, tid_v, stage_v, rows_v, sem):
        wid = lax.axis_index("s") * NUM_CORES + lax.axis_index("c")
        base = wid * b_per_w
        pltpu.sync_copy(idx_hbm.at[pl.ds(base, b_per_w)], idx_v)

        def to_row_ids(g, carry):
            v = idx_v[pl.ds(g * LANES, LANES)]
            tid_v[g // (CHUNK // LANES),
                  pl.ds((g % (CHUNK // LANES)) * LANES, LANES)] = (
                      jnp.where(v < half, v, v - half))
            return carry

        lax.fori_loop(0, b_per_w // LANES, to_row_ids, 0)

        def do_chunk(ch, carry):
            pltpu.async_copy(table_hbm.at[tid_v.at[ch]], stage_v, sem).wait()

            def select(g, c2):
                v = idx_v[pl.ds(ch * CHUNK + g * LANES, LANES)]
                for j in range(LANES):
                    hsel = jnp.where(v[j] < half, 0, dim)
                    p = g * LANES + j
                    for cg in range(col_groups):
                        rows_v[ch * CHUNK + p, pl.ds(cg * LANES, LANES)] = (
                            stage_v[p, pl.ds(hsel + cg * LANES, LANES)]
                        )
                return c2

            lax.fori_loop(0, CHUNK // LANES, select, 0)
            return carry

        lax.fori_loop(0, n_chunks, do_chunk, 0)
        pltpu.sync_copy(rows_v, out_hbm.at[pl.ds(base, b_per_w)])

    return k


def kernel(idx, vectors):
    batch = idx.shape[0]
    rows, dim = vectors.shape
    packed = _make_pack(rows, dim)(vectors, vectors)
    gather = _make_gather(batch, rows // 2, dim)
    return gather(packed, idx.astype(jnp.int32))


# speedup vs baseline: 2.2151x; 2.2151x over previous
import functools

import jax
import jax.numpy as jnp
from jax import lax
from jax.experimental import pallas as pl
from jax.experimental.pallas import tpu as pltpu
from jax.experimental.pallas import tpu_sc as plsc

NUM_CORES = 2
NUM_SUBCORES = 16
NUM_WORKERS = NUM_CORES * NUM_SUBCORES
LANES = 16
NSEM = 4


def _make_gather(batch, dim):
    assert batch % (NUM_WORKERS * LANES) == 0
    b_per_w = batch // NUM_WORKERS
    mesh = plsc.VectorSubcoreMesh(core_axis_name="c", subcore_axis_name="s")

    @functools.partial(
        pl.kernel,
        mesh=mesh,
        out_type=jax.ShapeDtypeStruct((batch, dim), jnp.float32),
        scratch_types=[
            pltpu.VMEM((b_per_w,), jnp.int32),
            pltpu.VMEM((b_per_w, dim), jnp.float32),
            [pltpu.SemaphoreType.DMA] * NSEM,
        ],
    )
    def k(table_hbm, idx_hbm, out_hbm, idx_v, rows_v, sems):
        wid = lax.axis_index("s") * NUM_CORES + lax.axis_index("c")
        base = wid * b_per_w
        pltpu.sync_copy(idx_hbm.at[pl.ds(base, b_per_w)], idx_v)

        def chunk(g, carry):
            vec = idx_v[pl.ds(g * LANES, LANES)]
            for j in range(LANES):
                row = vec[j]
                pltpu.async_copy(
                    table_hbm.at[pl.ds(row, 1)],
                    rows_v.at[pl.ds(g * LANES + j, 1)],
                    sems[j % NSEM],
                )
            return carry

        lax.fori_loop(0, b_per_w // LANES, chunk, 0)
        share = b_per_w // NSEM
        for q in range(NSEM):
            pltpu.make_async_copy(
                table_hbm.at[pl.ds(0, share)],
                rows_v.at[pl.ds(q * share, share)],
                sems[q],
            ).wait()
        pltpu.sync_copy(rows_v, out_hbm.at[pl.ds(base, b_per_w)])

    return k


def kernel(idx, vectors):
    batch = idx.shape[0]
    dim = vectors.shape[1]
    gather = _make_gather(batch, dim)
    return gather(vectors, idx.astype(jnp.int32))
